# baseline (speedup 1.0000x reference)
import functools

import jax
import jax.numpy as jnp
from jax import lax
from jax.experimental import pallas as pl
from jax.experimental.pallas import tpu as pltpu
from jax.experimental.pallas import tpu_sc as plsc

OUT_DIM = 64
PAD_DIM = 128
W = 256


@functools.lru_cache(maxsize=None)
def _make_gather(NB, T):
    info = plsc.get_sparse_core_info()
    NC, NS = info.num_cores, info.num_subcores
    NW = NC * NS
    b_per_w = NB // NW
    assert NB % NW == 0 and b_per_w == 2 * W

    mesh = plsc.VectorSubcoreMesh(core_axis_name="c", subcore_axis_name="s")

    @functools.partial(
        pl.kernel,
        mesh=mesh,
        out_type=jax.ShapeDtypeStruct((T, OUT_DIM, NB), jnp.float32),
        scratch_types=[
            pltpu.VMEM((W,), jnp.int32),
            pltpu.VMEM((W,), jnp.int32),
            pltpu.VMEM((2, W, PAD_DIM), jnp.float32),
            pltpu.VMEM((2, OUT_DIM, W), jnp.float32),
            pltpu.SemaphoreType.DMA,
            pltpu.SemaphoreType.DMA,
            pltpu.SemaphoreType.DMA,
            pltpu.SemaphoreType.DMA,
        ],
        compiler_params=pltpu.CompilerParams(use_tc_tiling_on_sc=True, needs_layout_passes=False),
    )
    def k(idx_hbm, table_hbm, out_hbm, idx0, idx1, g_buf, t_buf,
          gs0, gs1, os0, os1):
        wid = lax.axis_index("s") * NC + lax.axis_index("c")
        b0 = wid * b_per_w
        idxs = [idx0, idx1]
        gsem = [gs0, gs1]
        osem = [os0, os1]
        iota = lax.iota(jnp.int32, 16)

        def load_idx(t, h, s):
            off = t * NB + b0 + h * W
            pltpu.sync_copy(idx_hbm.at[pl.ds(off, W)], idxs[s])

        def start_gather(s):
            pltpu.async_copy(table_hbm.at[idxs[s]], g_buf.at[s], gsem[s])

        def wait_gather(s):
            pltpu.make_async_copy(
                table_hbm.at[idxs[s]], g_buf.at[s], gsem[s]
            ).wait()

        def out_slice(t, h):
            return out_hbm.at[t, pl.ds(0, OUT_DIM), pl.ds(b0 + h * W, W)]

        def start_store(t, h, s):
            pltpu.async_copy(t_buf.at[s], out_slice(t, h), osem[s])

        def wait_store(t, h, s):
            pltpu.make_async_copy(t_buf.at[s], out_slice(t, h), osem[s]).wait()

        def transpose(s):
            src = g_buf.at[s]

            def cbody(cc, carry):
                for dc in range(4):
                    c = cc * 4 + dc
                    colv = jnp.full((16,), 0, jnp.int32) + c
                    for j0 in range(0, W, 16):
                        v = plsc.load_gather(src, [iota + j0, colv])
                        t_buf[s, c, pl.ds(j0, 16)] = v
                return carry

            lax.fori_loop(0, OUT_DIM // 4, cbody, 0)

        load_idx(0, 0, 0)
        start_gather(0)
        load_idx(0, 1, 1)
        start_gather(1)
        wait_gather(0)
        transpose(0)
        start_store(0, 0, 0)
        load_idx(1, 0, 0)
        start_gather(0)
        wait_gather(1)
        transpose(1)
        start_store(0, 1, 1)

        def body(u, carry):
            load_idx(u, 1, 1)
            start_gather(1)
            wait_gather(0)
            wait_store(u - 1, 0, 0)
            transpose(0)
            start_store(u, 0, 0)
            load_idx(u + 1, 0, 0)
            start_gather(0)
            wait_gather(1)
            wait_store(u - 1, 1, 1)
            transpose(1)
            start_store(u, 1, 1)
            return carry

        lax.fori_loop(1, T - 1, body, 0)

        tl = T - 1
        load_idx(tl, 1, 1)
        start_gather(1)
        wait_gather(0)
        wait_store(tl - 1, 0, 0)
        transpose(0)
        start_store(tl, 0, 0)
        wait_gather(1)
        wait_store(tl - 1, 1, 1)
        transpose(1)
        start_store(tl, 1, 1)
        wait_store(tl, 0, 0)
        wait_store(tl, 1, 1)

    return k


def kernel(x, table):
    NB, T = x.shape
    idx = x.T.reshape(NB * T).astype(jnp.int32)
    table_pad = jnp.pad(table, ((0, 0), (0, PAD_DIM - OUT_DIM)))
    out = _make_gather(NB, T)(idx, table_pad)
    return jnp.transpose(out, (2, 0, 1))


# speedup vs baseline: 1.2351x; 1.2351x over previous
import functools

import jax
import jax.numpy as jnp
from jax import lax
from jax.experimental import pallas as pl
from jax.experimental.pallas import tpu as pltpu
from jax.experimental.pallas import tpu_sc as plsc

OUT_DIM = 64
PAD_DIM = 128
W = 256


@functools.lru_cache(maxsize=None)
def _make_gather(NB, T):
    info = plsc.get_sparse_core_info()
    NC, NS = info.num_cores, info.num_subcores
    NW = NC * NS
    b_per_w = NB // NW
    assert NB % NW == 0 and b_per_w == 2 * W

    mesh = plsc.VectorSubcoreMesh(core_axis_name="c", subcore_axis_name="s")

    @functools.partial(
        pl.kernel,
        mesh=mesh,
        out_type=jax.ShapeDtypeStruct((T, OUT_DIM, NB), jnp.float32),
        scratch_types=[
            pltpu.VMEM((W,), jnp.int32),
            pltpu.VMEM((W,), jnp.int32),
            pltpu.VMEM((2, W, PAD_DIM), jnp.float32),
            pltpu.VMEM((2, OUT_DIM, W), jnp.float32),
            pltpu.SemaphoreType.DMA,
            pltpu.SemaphoreType.DMA,
            pltpu.SemaphoreType.DMA,
            pltpu.SemaphoreType.DMA,
        ],
        compiler_params=pltpu.CompilerParams(use_tc_tiling_on_sc=True, needs_layout_passes=False),
    )
    def k(idx_hbm, table_hbm, out_hbm, idx0, idx1, g_buf, t_buf,
          gs0, gs1, os0, os1):
        wid = lax.axis_index("s") * NC + lax.axis_index("c")
        b0 = wid * b_per_w
        idxs = [idx0, idx1]
        gsem = [gs0, gs1]
        osem = [os0, os1]
        iota = lax.iota(jnp.int32, 16)

        def load_idx(t, h, s):
            off = t * NB + b0 + h * W
            pltpu.sync_copy(idx_hbm.at[pl.ds(off, W)], idxs[s])

        def start_gather(s):
            pltpu.async_copy(table_hbm.at[idxs[s]], g_buf.at[s], gsem[s])

        def wait_gather(s):
            pltpu.make_async_copy(
                table_hbm.at[idxs[s]], g_buf.at[s], gsem[s]
            ).wait()

        def out_slice(t, h):
            return out_hbm.at[t, pl.ds(0, OUT_DIM), pl.ds(b0 + h * W, W)]

        def start_store(t, h, s):
            pltpu.async_copy(t_buf.at[s], out_slice(t, h), osem[s])

        def wait_store(t, h, s):
            pltpu.make_async_copy(t_buf.at[s], out_slice(t, h), osem[s]).wait()

        def transpose(s):
            src = g_buf.at[s]

            def cbody(cc, carry):
                for dc in range(4):
                    c = cc * 4 + dc
                    colv = jnp.full((16,), 0, jnp.int32) + c
                    for j0 in range(0, W, 128):
                        vs = [
                            plsc.load_gather(src, [iota + (j0 + 16 * k), colv])
                            for k in range(8)
                        ]
                        for k in range(8):
                            t_buf[s, c, pl.ds(j0 + 16 * k, 16)] = vs[k]
                return carry

            lax.fori_loop(0, OUT_DIM // 4, cbody, 0)

        load_idx(0, 0, 0)
        start_gather(0)
        load_idx(0, 1, 1)
        start_gather(1)
        wait_gather(0)
        transpose(0)
        start_store(0, 0, 0)
        load_idx(1, 0, 0)
        start_gather(0)
        wait_gather(1)
        transpose(1)
        start_store(0, 1, 1)

        def body(u, carry):
            load_idx(u, 1, 1)
            start_gather(1)
            wait_gather(0)
            wait_store(u - 1, 0, 0)
            transpose(0)
            start_store(u, 0, 0)
            load_idx(u + 1, 0, 0)
            start_gather(0)
            wait_gather(1)
            wait_store(u - 1, 1, 1)
            transpose(1)
            start_store(u, 1, 1)
            return carry

        lax.fori_loop(1, T - 1, body, 0)

        tl = T - 1
        load_idx(tl, 1, 1)
        start_gather(1)
        wait_gather(0)
        wait_store(tl - 1, 0, 0)
        transpose(0)
        start_store(tl, 0, 0)
        wait_gather(1)
        wait_store(tl - 1, 1, 1)
        transpose(1)
        start_store(tl, 1, 1)
        wait_store(tl, 0, 0)
        wait_store(tl, 1, 1)

    return k


def kernel(x, table):
    NB, T = x.shape
    idx = x.T.reshape(NB * T).astype(jnp.int32)
    table_pad = jnp.pad(table, ((0, 0), (0, PAD_DIM - OUT_DIM)))
    out = _make_gather(NB, T)(idx, table_pad)
    return jnp.transpose(out, (2, 0, 1))


# speedup vs baseline: 2.0369x; 1.6492x over previous
import functools

import jax
import jax.numpy as jnp
from jax import lax
from jax.experimental import pallas as pl
from jax.experimental.pallas import tpu as pltpu
from jax.experimental.pallas import tpu_sc as plsc

OUT_DIM = 64
PAD_DIM = 128
W = 256


@functools.lru_cache(maxsize=None)
def _make_gather(NB, T):
    info = plsc.get_sparse_core_info()
    NC, NS = info.num_cores, info.num_subcores
    NW = NC * NS
    b_per_w = NB // NW
    assert NB % NW == 0 and b_per_w == 2 * W

    mesh = plsc.VectorSubcoreMesh(core_axis_name="c", subcore_axis_name="s")

    @functools.partial(
        pl.kernel,
        mesh=mesh,
        out_type=jax.ShapeDtypeStruct((T, OUT_DIM, NB), jnp.float32),
        scratch_types=[
            pltpu.VMEM((W,), jnp.int32),
            pltpu.VMEM((W,), jnp.int32),
            pltpu.VMEM((2, W, PAD_DIM), jnp.float32),
            pltpu.VMEM((2, OUT_DIM, W), jnp.float32),
            pltpu.SemaphoreType.DMA,
            pltpu.SemaphoreType.DMA,
            pltpu.SemaphoreType.DMA,
            pltpu.SemaphoreType.DMA,
        ],
        compiler_params=pltpu.CompilerParams(use_tc_tiling_on_sc=True, needs_layout_passes=False),
    )
    def k(idx_hbm, table_hbm, out_hbm, idx0, idx1, g_buf, t_buf,
          gs0, gs1, os0, os1):
        wid = lax.axis_index("s") * NC + lax.axis_index("c")
        b0 = wid * b_per_w
        idxs = [idx0, idx1]
        gsem = [gs0, gs1]
        osem = [os0, os1]
        iota = lax.iota(jnp.int32, 16)

        def load_idx(t, h, s):
            off = t * NB + b0 + h * W
            pltpu.sync_copy(idx_hbm.at[pl.ds(off, W)], idxs[s])

        def start_gather(s):
            pltpu.async_copy(table_hbm.at[idxs[s]], g_buf.at[s], gsem[s])

        def wait_gather(s):
            pltpu.make_async_copy(
                table_hbm.at[idxs[s]], g_buf.at[s], gsem[s]
            ).wait()

        def out_slice(t, h):
            return out_hbm.at[t, pl.ds(0, OUT_DIM), pl.ds(b0 + h * W, W)]

        def start_store(t, h, s):
            pltpu.async_copy(t_buf.at[s], out_slice(t, h), osem[s])

        def wait_store(t, h, s):
            pltpu.make_async_copy(t_buf.at[s], out_slice(t, h), osem[s]).wait()

        rots = [lax.rem(iota + r, jnp.int32(16)) for r in range(16)]

        def transpose(s):
            src = g_buf.at[s]
            dst = t_buf.at[s]

            def jbody(jj, carry):
                rows = iota + jj * 16
                for cc in range(OUT_DIM // 16):
                    for r in range(16):
                        cols = rots[r] + (16 * cc)
                        v = plsc.load_gather(src, [rows, cols])
                        plsc.store_scatter(dst, [cols, rows], v)
                return carry

            lax.fori_loop(0, W // 16, jbody, 0)

        load_idx(0, 0, 0)
        start_gather(0)
        load_idx(0, 1, 1)
        start_gather(1)
        wait_gather(0)
        transpose(0)
        start_store(0, 0, 0)
        load_idx(1, 0, 0)
        start_gather(0)
        wait_gather(1)
        transpose(1)
        start_store(0, 1, 1)

        def body(u, carry):
            load_idx(u, 1, 1)
            start_gather(1)
            wait_gather(0)
            wait_store(u - 1, 0, 0)
            transpose(0)
            start_store(u, 0, 0)
            load_idx(u + 1, 0, 0)
            start_gather(0)
            wait_gather(1)
            wait_store(u - 1, 1, 1)
            transpose(1)
            start_store(u, 1, 1)
            return carry

        lax.fori_loop(1, T - 1, body, 0)

        tl = T - 1
        load_idx(tl, 1, 1)
        start_gather(1)
        wait_gather(0)
        wait_store(tl - 1, 0, 0)
        transpose(0)
        start_store(tl, 0, 0)
        wait_gather(1)
        wait_store(tl - 1, 1, 1)
        transpose(1)
        start_store(tl, 1, 1)
        wait_store(tl, 0, 0)
        wait_store(tl, 1, 1)

    return k


def kernel(x, table):
    NB, T = x.shape
    idx = x.T.reshape(NB * T).astype(jnp.int32)
    table_pad = jnp.pad(table, ((0, 0), (0, PAD_DIM - OUT_DIM)))
    out = _make_gather(NB, T)(idx, table_pad)
    return jnp.transpose(out, (2, 0, 1))


# speedup vs baseline: 3.6384x; 1.7862x over previous
import functools

import jax
import jax.numpy as jnp
from jax import lax
from jax.experimental import pallas as pl
from jax.experimental.pallas import tpu as pltpu
from jax.experimental.pallas import tpu_sc as plsc

OUT_DIM = 64
PAD_DIM = 128
W = 256


@functools.lru_cache(maxsize=None)
def _make_gather(NB, T):
    info = plsc.get_sparse_core_info()
    NC, NS = info.num_cores, info.num_subcores
    NW = NC * NS
    b_per_w = NB // NW
    assert NB % NW == 0 and b_per_w == 2 * W

    mesh = plsc.VectorSubcoreMesh(core_axis_name="c", subcore_axis_name="s")

    @functools.partial(
        pl.kernel,
        mesh=mesh,
        out_type=jax.ShapeDtypeStruct((T, OUT_DIM, NB), jnp.float32),
        scratch_types=[
            pltpu.VMEM((W,), jnp.int32),
            pltpu.VMEM((W,), jnp.int32),
            pltpu.VMEM((2, W, PAD_DIM), jnp.float32),
            pltpu.VMEM((2, OUT_DIM, W), jnp.float32),
            pltpu.SemaphoreType.DMA,
            pltpu.SemaphoreType.DMA,
            pltpu.SemaphoreType.DMA,
            pltpu.SemaphoreType.DMA,
        ],
        compiler_params=pltpu.CompilerParams(use_tc_tiling_on_sc=True, needs_layout_passes=False),
    )
    def k(idx_hbm, table_hbm, out_hbm, idx0, idx1, g_buf, t_buf,
          gs0, gs1, os0, os1):
        wid = lax.axis_index("s") * NC + lax.axis_index("c")
        b0 = wid * b_per_w
        idxs = [idx0, idx1]
        gsem = [gs0, gs1]
        osem = [os0, os1]
        iota = lax.iota(jnp.int32, 16)

        def load_idx(t, h, s):
            off = t * NB + b0 + h * W
            pltpu.sync_copy(idx_hbm.at[pl.ds(off, W)], idxs[s])

        def start_gather(s):
            pltpu.async_copy(table_hbm.at[idxs[s]], g_buf.at[s], gsem[s])

        def wait_gather(s):
            pltpu.make_async_copy(
                table_hbm.at[idxs[s]], g_buf.at[s], gsem[s]
            ).wait()

        def out_slice(t, h):
            return out_hbm.at[t, pl.ds(0, OUT_DIM), pl.ds(b0 + h * W, W)]

        def start_store(t, h, s):
            pltpu.async_copy(t_buf.at[s], out_slice(t, h), osem[s])

        def wait_store(t, h, s):
            pltpu.make_async_copy(t_buf.at[s], out_slice(t, h), osem[s]).wait()

        rots = [lax.rem(iota + r, jnp.int32(16)) for r in range(16)]

        def transpose(s):
            src = g_buf.at[s]
            dst = t_buf.at[s]

            def jbody(jj, carry):
                rows = iota + jj * 16
                for cc in range(OUT_DIM // 16):
                    cols = [rots[r] + (16 * cc) for r in range(16)]
                    for half in range(0, 16, 8):
                        vs = [
                            plsc.load_gather(src, [rows, cols[half + k]])
                            for k in range(8)
                        ]
                        for k in range(8):
                            plsc.store_scatter(
                                dst, [cols[half + k], rows], vs[k]
                            )
                return carry

            lax.fori_loop(0, W // 16, jbody, 0)

        load_idx(0, 0, 0)
        start_gather(0)

        def body(u, carry):
            load_idx(u, 1, 1)
            start_gather(1)
            wait_gather(0)

            @pl.when(u >= 1)
            def _():
                wait_store(u - 1, 0, 0)

            transpose(0)
            start_store(u, 0, 0)

            @pl.when(u + 1 <= T - 1)
            def _():
                load_idx(u + 1, 0, 0)
                start_gather(0)

            wait_gather(1)

            @pl.when(u >= 1)
            def _():
                wait_store(u - 1, 1, 1)

            transpose(1)
            start_store(u, 1, 1)
            return carry

        lax.fori_loop(0, T, body, 0)
        wait_store(T - 1, 0, 0)
        wait_store(T - 1, 1, 1)

    return k


def kernel(x, table):
    NB, T = x.shape
    idx = x.T.reshape(NB * T).astype(jnp.int32)
    table_pad = jnp.pad(table, ((0, 0), (0, PAD_DIM - OUT_DIM)))
    out = _make_gather(NB, T)(idx, table_pad)
    return jnp.transpose(out, (2, 0, 1))


# speedup vs baseline: 4.5769x; 1.2579x over previous
import functools

import jax
import jax.numpy as jnp
from jax import lax
from jax.experimental import pallas as pl
from jax.experimental.pallas import tpu as pltpu
from jax.experimental.pallas import tpu_sc as plsc

OUT_DIM = 64
PAD_DIM = 128
W = 256


@functools.lru_cache(maxsize=None)
def _make_gather(NB, T):
    info = plsc.get_sparse_core_info()
    NC, NS = info.num_cores, info.num_subcores
    NW = NC * NS
    b_per_w = NB // NW
    assert NB % NW == 0 and b_per_w == 2 * W

    mesh = plsc.VectorSubcoreMesh(core_axis_name="c", subcore_axis_name="s")

    @functools.partial(
        pl.kernel,
        mesh=mesh,
        out_type=jax.ShapeDtypeStruct((T, OUT_DIM, NB), jnp.float32),
        scratch_types=[
            pltpu.VMEM((W,), jnp.int32),
            pltpu.VMEM((W,), jnp.int32),
            pltpu.VMEM((2, W, PAD_DIM), jnp.float32),
            pltpu.VMEM((2, OUT_DIM, W), jnp.float32),
            pltpu.SemaphoreType.DMA,
            pltpu.SemaphoreType.DMA,
            pltpu.SemaphoreType.DMA,
            pltpu.SemaphoreType.DMA,
        ],
        compiler_params=pltpu.CompilerParams(use_tc_tiling_on_sc=True, needs_layout_passes=False),
    )
    def k(idx_hbm, table_hbm, out_hbm, idx0, idx1, g_buf, t_buf,
          gs0, gs1, os0, os1):
        wid = lax.axis_index("s") * NC + lax.axis_index("c")
        b0 = wid * b_per_w
        idxs = [idx0, idx1]
        gsem = [gs0, gs1]
        osem = [os0, os1]
        iota = lax.iota(jnp.int32, 16)

        def load_idx(t, h, s):
            off = t * NB + b0 + h * W
            pltpu.sync_copy(idx_hbm.at[pl.ds(off, W)], idxs[s])

        def start_gather(s):
            pltpu.async_copy(table_hbm.at[idxs[s]], g_buf.at[s], gsem[s])

        def wait_gather(s):
            pltpu.make_async_copy(
                table_hbm.at[idxs[s]], g_buf.at[s], gsem[s]
            ).wait()

        def out_slice(t, h):
            return out_hbm.at[t, pl.ds(0, OUT_DIM), pl.ds(b0 + h * W, W)]

        def start_store(t, h, s):
            pltpu.async_copy(t_buf.at[s], out_slice(t, h), osem[s])

        def wait_store(t, h, s):
            pltpu.make_async_copy(t_buf.at[s], out_slice(t, h), osem[s]).wait()

        rots = [lax.rem(iota + r, jnp.int32(16)) for r in range(16)]

        def transpose(s):
            src = g_buf.at[s]
            dst = t_buf.at[s]

            def jbody(jj, carry):
                rows = iota + jj * 16
                for cc in range(OUT_DIM // 16):
                    cols = [rots[r] + (16 * cc) for r in range(16)]
                    vs = [
                        plsc.load_gather(src, [rows, cols[r]])
                        for r in range(16)
                    ]
                    for r in range(16):
                        plsc.store_scatter(dst, [cols[r], rows], vs[r])
                return carry

            lax.fori_loop(0, W // 16, jbody, 0)

        load_idx(0, 0, 0)
        start_gather(0)

        def body(u, carry):
            load_idx(u, 1, 1)
            start_gather(1)
            wait_gather(0)

            @pl.when(u >= 1)
            def _():
                wait_store(u - 1, 0, 0)

            transpose(0)
            start_store(u, 0, 0)

            @pl.when(u + 1 <= T - 1)
            def _():
                load_idx(u + 1, 0, 0)
                start_gather(0)

            wait_gather(1)

            @pl.when(u >= 1)
            def _():
                wait_store(u - 1, 1, 1)

            transpose(1)
            start_store(u, 1, 1)
            return carry

        lax.fori_loop(0, T, body, 0)
        wait_store(T - 1, 0, 0)
        wait_store(T - 1, 1, 1)

    return k


def kernel(x, table):
    NB, T = x.shape
    idx = x.T.reshape(NB * T).astype(jnp.int32)
    table_pad = jnp.pad(table, ((0, 0), (0, PAD_DIM - OUT_DIM)))
    out = _make_gather(NB, T)(idx, table_pad)
    return jnp.transpose(out, (2, 0, 1))
